# baseline (speedup 1.0000x reference)
import functools

import jax
import jax.numpy as jnp
from jax import lax
from jax.experimental import pallas as pl
from jax.experimental.pallas import tpu as pltpu
from jax.experimental.pallas import tpu_sc as plsc

N_NODES = 10000
N_EDGES = 320000
D = 128

NC, NS, L = 2, 16, 16
NW = NC * NS
NPAD = 10240
EPW = 10240
CHUNK = 128
NCHUNK = EPW // CHUNK
DD = 16
ROWS_PER_SUB = NPAD // NS
STAGE = 64

_MESH = plsc.VectorSubcoreMesh(
    core_axis_name="c", subcore_axis_name="s", num_cores=NC, num_subcores=NS
)


@functools.partial(
    pl.kernel,
    out_type=jax.ShapeDtypeStruct((NC, NPAD, D), jnp.float32),
    mesh=_MESH,
    scratch_types=[
        pltpu.VMEM((NCHUNK, CHUNK), jnp.int32),
        pltpu.VMEM((CHUNK, D), jnp.float32),
        pltpu.VMEM((STAGE, D), jnp.float32),
        pltpu.VMEM_SHARED((NPAD, D), jnp.float32),
        pltpu.SemaphoreType.DMA,
    ],
)
def _deg_kernel(dst_hbm, out_hbm, dst_v, ones_v, stage_v, deg_sh, sems):
    c = lax.axis_index("c")
    s = lax.axis_index("s")
    wid = s * NC + c

    def _zrow(i, carry):
        for k in range(D // L):
            stage_v[i, pl.ds(k * L, L)] = jnp.zeros((L,), jnp.float32)
        return carry

    lax.fori_loop(0, STAGE, _zrow, 0)

    def _orow(i, carry):
        for k in range(D // L):
            ones_v[i, pl.ds(k * L, L)] = jnp.ones((L,), jnp.float32)
        return carry

    lax.fori_loop(0, CHUNK, _orow, 0)
    for t in range(ROWS_PER_SUB // STAGE):
        pltpu.sync_copy(
            stage_v, deg_sh.at[pl.ds(s * ROWS_PER_SUB + t * STAGE, STAGE)]
        )
    plsc.subcore_barrier()

    pltpu.sync_copy(dst_hbm.at[wid], dst_v)

    W = 8

    def _edges(t, carry):
        for k in range(W):
            pltpu.async_copy(ones_v, deg_sh.at[dst_v.at[t * W + k]], sems, add=True)
        for k in range(W):
            pltpu.make_async_copy(ones_v, deg_sh.at[dst_v.at[0]], sems).wait()
        return carry

    lax.fori_loop(0, NCHUNK // W, _edges, 0)
    plsc.subcore_barrier()

    for t in range(ROWS_PER_SUB // STAGE):
        r0 = s * ROWS_PER_SUB + t * STAGE
        pltpu.sync_copy(deg_sh.at[pl.ds(r0, STAGE)], stage_v)
        pltpu.sync_copy(stage_v, out_hbm.at[c, pl.ds(r0, STAGE)])


GROUPS = 5
GCHUNK = NCHUNK // GROUPS
HALF = GCHUNK // 2


@functools.partial(
    pl.kernel,
    out_type=jax.ShapeDtypeStruct((NC, NPAD, D), jnp.float32),
    mesh=_MESH,
    scratch_types=[
        pltpu.VMEM((GCHUNK, CHUNK), jnp.int32),
        pltpu.VMEM((GCHUNK, CHUNK), jnp.int32),
        pltpu.VMEM((CHUNK, D), jnp.float32),
        pltpu.VMEM((CHUNK, D), jnp.float32),
        pltpu.VMEM((STAGE, D), jnp.float32),
        pltpu.VMEM_SHARED((NPAD, D), jnp.float32),
        pltpu.SemaphoreType.DMA,
        pltpu.SemaphoreType.DMA,
    ],
)
def _agg_kernel(y_hbm, src_hbm, dst_hbm, out_hbm,
                src_v, dst_v, rows0, rows1, stage_v, agg_sh,
                sem_g0, sem_g1):
    c = lax.axis_index("c")
    s = lax.axis_index("s")
    wid = s * NC + c

    def _zrow(i, carry):
        for k in range(D // L):
            stage_v[i, pl.ds(k * L, L)] = jnp.zeros((L,), jnp.float32)
        return carry

    lax.fori_loop(0, STAGE, _zrow, 0)
    for t in range(ROWS_PER_SUB // STAGE):
        pltpu.sync_copy(
            stage_v, agg_sh.at[pl.ds(s * ROWS_PER_SUB + t * STAGE, STAGE)]
        )
    plsc.subcore_barrier()

    def _group(g, carry):
        pltpu.sync_copy(src_hbm.at[wid, pl.ds(g * GCHUNK, GCHUNK)], src_v)
        pltpu.sync_copy(dst_hbm.at[wid, pl.ds(g * GCHUNK, GCHUNK)], dst_v)
        pltpu.async_copy(y_hbm.at[src_v.at[0]], rows0, sem_g0)
        pltpu.async_copy(y_hbm.at[src_v.at[1]], rows1, sem_g1)

        def _pair(jj, carry2):
            a = 2 * jj
            pltpu.make_async_copy(y_hbm.at[src_v.at[0]], rows0, sem_g0).wait()
            pltpu.sync_copy(rows0, agg_sh.at[dst_v.at[a]], add=True)

            @pl.when(jj < HALF - 1)
            def _():
                pltpu.async_copy(y_hbm.at[src_v.at[a + 2]], rows0, sem_g0)

            pltpu.make_async_copy(y_hbm.at[src_v.at[1]], rows1, sem_g1).wait()
            pltpu.sync_copy(rows1, agg_sh.at[dst_v.at[a + 1]], add=True)

            @pl.when(jj < HALF - 1)
            def _():
                pltpu.async_copy(y_hbm.at[src_v.at[a + 3]], rows1, sem_g1)

            return carry2

        lax.fori_loop(0, HALF, _pair, 0)
        return carry

    lax.fori_loop(0, GROUPS, _group, 0)
    plsc.subcore_barrier()

    for t in range(ROWS_PER_SUB // STAGE):
        r0 = s * ROWS_PER_SUB + t * STAGE
        pltpu.sync_copy(agg_sh.at[pl.ds(r0, STAGE)], stage_v)
        pltpu.sync_copy(stage_v, out_hbm.at[c, pl.ds(r0, STAGE)])


BM = 1024


def _dinv_block(degp_ref):
    deg = degp_ref[0, :, 0:1] + degp_ref[1, :, 0:1] + 1.0
    return lax.rsqrt(deg)


def _mm_body(x_ref, w_ref, hw_ref):
    hw_ref[...] = jnp.dot(x_ref[...], w_ref[...], preferred_element_type=jnp.float32)


def _scale_body(hw_ref, degp_ref, y_ref, dinv_ref):
    dinv = _dinv_block(degp_ref)
    y_ref[...] = hw_ref[...] * dinv
    dinv_ref[...] = dinv


def _mid_body(aggp_ref, y1_ref, dinv_ref, b_ref, w_ref, y2_ref):
    dinv = dinv_ref[...]
    t = (aggp_ref[0] + aggp_ref[1] + y1_ref[...]) * dinv + b_ref[...]
    h = jnp.maximum(t, 0.0)
    y2_ref[...] = jnp.dot(h, w_ref[...], preferred_element_type=jnp.float32) * dinv


def _out_body(aggp_ref, y2_ref, dinv_ref, b_ref, o_ref):
    dinv = dinv_ref[...]
    o_ref[...] = (aggp_ref[0] + aggp_ref[1] + y2_ref[...]) * dinv + b_ref[...]


_ROW = pl.BlockSpec((BM, D), lambda i: (i, 0))
_FULL = pl.BlockSpec((D, D), lambda i: (0, 0))
_DEGP = pl.BlockSpec((NC, BM, D), lambda i: (0, i, 0))
_AGGP = pl.BlockSpec((NC, BM, D), lambda i: (0, i, 0))
_BIAS = pl.BlockSpec((1, D), lambda i: (0, 0))
_DINV = pl.BlockSpec((BM, 1), lambda i: (i, 0))
_OUT = jax.ShapeDtypeStruct((NPAD, D), jnp.float32)
_DINV_OUT = jax.ShapeDtypeStruct((NPAD, 1), jnp.float32)
_GRID = (NPAD // BM,)

_mm = pl.pallas_call(
    _mm_body, grid=_GRID, out_shape=_OUT,
    in_specs=[_ROW, _FULL], out_specs=_ROW,
)
_scale = pl.pallas_call(
    _scale_body, grid=_GRID, out_shape=[_OUT, _DINV_OUT],
    in_specs=[_ROW, _DEGP], out_specs=[_ROW, _DINV],
)
_mid = pl.pallas_call(
    _mid_body, grid=_GRID, out_shape=_OUT,
    in_specs=[_AGGP, _ROW, _DINV, _BIAS, _FULL], out_specs=_ROW,
)
_out = pl.pallas_call(
    _out_body, grid=_GRID, out_shape=_OUT,
    in_specs=[_AGGP, _ROW, _DINV, _BIAS], out_specs=_ROW,
)


def kernel(x, edge_index, W1, b1, W2, b2):
    src = edge_index[0].astype(jnp.int32)
    dst = edge_index[1].astype(jnp.int32)
    npadded = NW * EPW - N_EDGES
    pad = N_NODES + (jnp.arange(npadded, dtype=jnp.int32) % (NPAD - N_NODES))
    src_r = jnp.concatenate([src, pad]).reshape(NW, NCHUNK, CHUNK)
    dst_r = jnp.concatenate([dst, pad]).reshape(NW, NCHUNK, CHUNK)
    x_pad = jnp.zeros((NPAD, D), jnp.float32).at[:N_NODES].set(x)

    hw1 = _mm(x_pad, W1)
    degp = _deg_kernel(dst_r)
    y1, dinv = _scale(hw1, degp)
    agg1 = _agg_kernel(y1, src_r, dst_r)
    y2 = _mid(agg1, y1, dinv, b1.reshape(1, D), W2)
    agg2 = _agg_kernel(y2, src_r, dst_r)
    out = _out(agg2, y2, dinv, b2.reshape(1, D))
    return out[:N_NODES]


# speedup vs baseline: 1.0265x; 1.0265x over previous
import functools

import jax
import jax.numpy as jnp
from jax import lax
from jax.experimental import pallas as pl
from jax.experimental.pallas import tpu as pltpu
from jax.experimental.pallas import tpu_sc as plsc

N_NODES = 10000
N_EDGES = 320000
D = 128

NC, NS, L = 2, 16, 16
NW = NC * NS
NPAD = 10240
EPW = 10240
CHUNK = 128
NCHUNK = EPW // CHUNK
DD = 16
ROWS_PER_SUB = NPAD // NS
STAGE = 32

_MESH = plsc.VectorSubcoreMesh(
    core_axis_name="c", subcore_axis_name="s", num_cores=NC, num_subcores=NS
)


@functools.partial(
    pl.kernel,
    out_type=jax.ShapeDtypeStruct((NC, NPAD, D), jnp.float32),
    mesh=_MESH,
    scratch_types=[
        pltpu.VMEM((NCHUNK, CHUNK), jnp.int32),
        pltpu.VMEM((CHUNK, D), jnp.float32),
        pltpu.VMEM((STAGE, D), jnp.float32),
        pltpu.VMEM_SHARED((NPAD, D), jnp.float32),
        pltpu.SemaphoreType.DMA,
    ],
)
def _deg_kernel(dst_hbm, out_hbm, dst_v, ones_v, stage_v, deg_sh, sems):
    c = lax.axis_index("c")
    s = lax.axis_index("s")
    wid = s * NC + c

    def _zrow(i, carry):
        for k in range(D // L):
            stage_v[i, pl.ds(k * L, L)] = jnp.zeros((L,), jnp.float32)
        return carry

    lax.fori_loop(0, STAGE, _zrow, 0)

    def _orow(i, carry):
        for k in range(D // L):
            ones_v[i, pl.ds(k * L, L)] = jnp.ones((L,), jnp.float32)
        return carry

    lax.fori_loop(0, CHUNK, _orow, 0)
    for t in range(ROWS_PER_SUB // STAGE):
        pltpu.sync_copy(
            stage_v, deg_sh.at[pl.ds(s * ROWS_PER_SUB + t * STAGE, STAGE)]
        )
    plsc.subcore_barrier()

    pltpu.sync_copy(dst_hbm.at[wid], dst_v)

    W = 8

    def _edges(t, carry):
        for k in range(W):
            pltpu.async_copy(ones_v, deg_sh.at[dst_v.at[t * W + k]], sems, add=True)
        for k in range(W):
            pltpu.make_async_copy(ones_v, deg_sh.at[dst_v.at[0]], sems).wait()
        return carry

    lax.fori_loop(0, NCHUNK // W, _edges, 0)
    plsc.subcore_barrier()

    for t in range(ROWS_PER_SUB // STAGE):
        r0 = s * ROWS_PER_SUB + t * STAGE
        pltpu.sync_copy(deg_sh.at[pl.ds(r0, STAGE)], stage_v)
        pltpu.sync_copy(stage_v, out_hbm.at[c, pl.ds(r0, STAGE)])


GROUPS = 2
GCHUNK = NCHUNK // GROUPS
HALF = GCHUNK // 2


@functools.partial(
    pl.kernel,
    out_type=jax.ShapeDtypeStruct((NC, NPAD, D), jnp.float32),
    mesh=_MESH,
    scratch_types=[
        pltpu.VMEM((GCHUNK, CHUNK), jnp.int32),
        pltpu.VMEM((GCHUNK, CHUNK), jnp.int32),
        pltpu.VMEM((CHUNK, D), jnp.float32),
        pltpu.VMEM((CHUNK, D), jnp.float32),
        pltpu.VMEM((STAGE, D), jnp.float32),
        pltpu.VMEM_SHARED((NPAD, D), jnp.float32),
        pltpu.SemaphoreType.DMA,
        pltpu.SemaphoreType.DMA,
    ],
)
def _agg_kernel(y_hbm, src_hbm, dst_hbm, out_hbm,
                src_v, dst_v, rows0, rows1, stage_v, agg_sh,
                sem_g0, sem_g1):
    c = lax.axis_index("c")
    s = lax.axis_index("s")
    wid = s * NC + c

    def _zrow(i, carry):
        for k in range(D // L):
            stage_v[i, pl.ds(k * L, L)] = jnp.zeros((L,), jnp.float32)
        return carry

    lax.fori_loop(0, STAGE, _zrow, 0)
    for t in range(ROWS_PER_SUB // STAGE):
        pltpu.sync_copy(
            stage_v, agg_sh.at[pl.ds(s * ROWS_PER_SUB + t * STAGE, STAGE)]
        )
    plsc.subcore_barrier()

    def _group(g, carry):
        pltpu.sync_copy(src_hbm.at[wid, pl.ds(g * GCHUNK, GCHUNK)], src_v)
        pltpu.sync_copy(dst_hbm.at[wid, pl.ds(g * GCHUNK, GCHUNK)], dst_v)
        pltpu.async_copy(y_hbm.at[src_v.at[0]], rows0, sem_g0)
        pltpu.async_copy(y_hbm.at[src_v.at[1]], rows1, sem_g1)

        def _pair(jj, carry2):
            a = 2 * jj
            pltpu.make_async_copy(y_hbm.at[src_v.at[0]], rows0, sem_g0).wait()
            pltpu.sync_copy(rows0, agg_sh.at[dst_v.at[a]], add=True)

            @pl.when(jj < HALF - 1)
            def _():
                pltpu.async_copy(y_hbm.at[src_v.at[a + 2]], rows0, sem_g0)

            pltpu.make_async_copy(y_hbm.at[src_v.at[1]], rows1, sem_g1).wait()
            pltpu.sync_copy(rows1, agg_sh.at[dst_v.at[a + 1]], add=True)

            @pl.when(jj < HALF - 1)
            def _():
                pltpu.async_copy(y_hbm.at[src_v.at[a + 3]], rows1, sem_g1)

            return carry2

        lax.fori_loop(0, HALF, _pair, 0)
        return carry

    lax.fori_loop(0, GROUPS, _group, 0)
    plsc.subcore_barrier()

    for t in range(ROWS_PER_SUB // STAGE):
        r0 = s * ROWS_PER_SUB + t * STAGE
        pltpu.sync_copy(agg_sh.at[pl.ds(r0, STAGE)], stage_v)
        pltpu.sync_copy(stage_v, out_hbm.at[c, pl.ds(r0, STAGE)])


BM = 1024


def _dinv_block(degp_ref):
    deg = degp_ref[0, :, 0:1] + degp_ref[1, :, 0:1] + 1.0
    return lax.rsqrt(deg)


def _mm_scale_body(x_ref, w_ref, degp_ref, y_ref, dinv_ref):
    hw = jnp.dot(x_ref[...], w_ref[...], preferred_element_type=jnp.float32)
    dinv = _dinv_block(degp_ref)
    y_ref[...] = hw * dinv
    dinv_ref[...] = dinv


def _mid_body(aggp_ref, y1_ref, dinv_ref, b_ref, w_ref, y2_ref):
    dinv = dinv_ref[...]
    t = (aggp_ref[0] + aggp_ref[1] + y1_ref[...]) * dinv + b_ref[...]
    h = jnp.maximum(t, 0.0)
    y2_ref[...] = jnp.dot(h, w_ref[...], preferred_element_type=jnp.float32) * dinv


def _out_body(aggp_ref, y2_ref, dinv_ref, b_ref, o_ref):
    dinv = dinv_ref[...]
    o_ref[...] = (aggp_ref[0] + aggp_ref[1] + y2_ref[...]) * dinv + b_ref[...]


_ROW = pl.BlockSpec((BM, D), lambda i: (i, 0))
_FULL = pl.BlockSpec((D, D), lambda i: (0, 0))
_DEGP = pl.BlockSpec((NC, BM, D), lambda i: (0, i, 0))
_AGGP = pl.BlockSpec((NC, BM, D), lambda i: (0, i, 0))
_BIAS = pl.BlockSpec((1, D), lambda i: (0, 0))
_DINV = pl.BlockSpec((BM, 1), lambda i: (i, 0))
_OUT = jax.ShapeDtypeStruct((NPAD, D), jnp.float32)
_DINV_OUT = jax.ShapeDtypeStruct((NPAD, 1), jnp.float32)
_GRID = (NPAD // BM,)

_mm_scale = pl.pallas_call(
    _mm_scale_body, grid=_GRID, out_shape=[_OUT, _DINV_OUT],
    in_specs=[_ROW, _FULL, _DEGP], out_specs=[_ROW, _DINV],
)
_mid = pl.pallas_call(
    _mid_body, grid=_GRID, out_shape=_OUT,
    in_specs=[_AGGP, _ROW, _DINV, _BIAS, _FULL], out_specs=_ROW,
)
_out = pl.pallas_call(
    _out_body, grid=_GRID, out_shape=_OUT,
    in_specs=[_AGGP, _ROW, _DINV, _BIAS], out_specs=_ROW,
)


def kernel(x, edge_index, W1, b1, W2, b2):
    src = edge_index[0].astype(jnp.int32)
    dst = edge_index[1].astype(jnp.int32)
    npadded = NW * EPW - N_EDGES
    pad = N_NODES + (jnp.arange(npadded, dtype=jnp.int32) % (NPAD - N_NODES))
    src_r = jnp.concatenate([src, pad]).reshape(NW, NCHUNK, CHUNK)
    dst_r = jnp.concatenate([dst, pad]).reshape(NW, NCHUNK, CHUNK)
    x_pad = jnp.zeros((NPAD, D), jnp.float32).at[:N_NODES].set(x)

    degp = _deg_kernel(dst_r)
    y1, dinv = _mm_scale(x_pad, W1, degp)
    agg1 = _agg_kernel(y1, src_r, dst_r)
    y2 = _mid(agg1, y1, dinv, b1.reshape(1, D), W2)
    agg2 = _agg_kernel(y2, src_r, dst_r)
    out = _out(agg2, y2, dinv, b2.reshape(1, D))
    return out[:N_NODES]


# speedup vs baseline: 1.0422x; 1.0154x over previous
import functools

import jax
import jax.numpy as jnp
from jax import lax
from jax.experimental import pallas as pl
from jax.experimental.pallas import tpu as pltpu
from jax.experimental.pallas import tpu_sc as plsc

N_NODES = 10000
N_EDGES = 320000
D = 128

NC, NS, L = 2, 16, 16
NW = NC * NS
NPAD = 10240
EPW = 10240
CHUNK = 128
NCHUNK = EPW // CHUNK
DD = 16
ROWS_PER_SUB = NPAD // NS
STAGE = 32

_MESH = plsc.VectorSubcoreMesh(
    core_axis_name="c", subcore_axis_name="s", num_cores=NC, num_subcores=NS
)


@functools.partial(
    pl.kernel,
    out_type=jax.ShapeDtypeStruct((NC, NPAD, D), jnp.float32),
    mesh=_MESH,
    scratch_types=[
        pltpu.VMEM((NCHUNK, CHUNK), jnp.int32),
        pltpu.VMEM((CHUNK, D), jnp.float32),
        pltpu.VMEM((STAGE, D), jnp.float32),
        pltpu.VMEM_SHARED((NPAD, D), jnp.float32),
        pltpu.SemaphoreType.DMA,
    ],
)
def _deg_kernel(dst_hbm, out_hbm, dst_v, ones_v, stage_v, deg_sh, sems):
    c = lax.axis_index("c")
    s = lax.axis_index("s")
    wid = s * NC + c

    def _zrow(i, carry):
        for k in range(D // L):
            stage_v[i, pl.ds(k * L, L)] = jnp.zeros((L,), jnp.float32)
        return carry

    lax.fori_loop(0, STAGE, _zrow, 0)

    def _orow(i, carry):
        for k in range(D // L):
            ones_v[i, pl.ds(k * L, L)] = jnp.ones((L,), jnp.float32)
        return carry

    lax.fori_loop(0, CHUNK, _orow, 0)
    for t in range(ROWS_PER_SUB // STAGE):
        pltpu.sync_copy(
            stage_v, deg_sh.at[pl.ds(s * ROWS_PER_SUB + t * STAGE, STAGE)]
        )
    plsc.subcore_barrier()

    pltpu.sync_copy(dst_hbm.at[wid], dst_v)

    W = 16

    def _edges(t, carry):
        for k in range(W):
            pltpu.async_copy(ones_v, deg_sh.at[dst_v.at[t * W + k]], sems, add=True)
        for k in range(W):
            pltpu.make_async_copy(ones_v, deg_sh.at[dst_v.at[0]], sems).wait()
        return carry

    lax.fori_loop(0, NCHUNK // W, _edges, 0)
    plsc.subcore_barrier()

    for t in range(ROWS_PER_SUB // STAGE):
        r0 = s * ROWS_PER_SUB + t * STAGE
        pltpu.sync_copy(deg_sh.at[pl.ds(r0, STAGE)], stage_v)
        pltpu.sync_copy(stage_v, out_hbm.at[c, pl.ds(r0, STAGE)])


GROUPS = 2
GCHUNK = NCHUNK // GROUPS
HALF = GCHUNK // 2


@functools.partial(
    pl.kernel,
    out_type=jax.ShapeDtypeStruct((NC, NPAD, D), jnp.float32),
    mesh=_MESH,
    scratch_types=[
        pltpu.VMEM((GCHUNK, CHUNK), jnp.int32),
        pltpu.VMEM((GCHUNK, CHUNK), jnp.int32),
        pltpu.VMEM((CHUNK, D), jnp.float32),
        pltpu.VMEM((CHUNK, D), jnp.float32),
        pltpu.VMEM((STAGE, D), jnp.float32),
        pltpu.VMEM_SHARED((NPAD, D), jnp.float32),
        pltpu.SemaphoreType.DMA,
        pltpu.SemaphoreType.DMA,
    ],
)
def _agg_kernel(y_hbm, src_hbm, dst_hbm, out_hbm,
                src_v, dst_v, rows0, rows1, stage_v, agg_sh,
                sem_g0, sem_g1):
    c = lax.axis_index("c")
    s = lax.axis_index("s")
    wid = s * NC + c

    def _zrow(i, carry):
        for k in range(D // L):
            stage_v[i, pl.ds(k * L, L)] = jnp.zeros((L,), jnp.float32)
        return carry

    lax.fori_loop(0, STAGE, _zrow, 0)
    for t in range(ROWS_PER_SUB // STAGE):
        pltpu.sync_copy(
            stage_v, agg_sh.at[pl.ds(s * ROWS_PER_SUB + t * STAGE, STAGE)]
        )
    plsc.subcore_barrier()

    def _group(g, carry):
        pltpu.sync_copy(src_hbm.at[wid, pl.ds(g * GCHUNK, GCHUNK)], src_v)
        pltpu.sync_copy(dst_hbm.at[wid, pl.ds(g * GCHUNK, GCHUNK)], dst_v)
        pltpu.async_copy(y_hbm.at[src_v.at[0]], rows0, sem_g0)
        pltpu.async_copy(y_hbm.at[src_v.at[1]], rows1, sem_g1)

        def _pair(jj, carry2):
            a = 2 * jj
            pltpu.make_async_copy(y_hbm.at[src_v.at[0]], rows0, sem_g0).wait()
            pltpu.sync_copy(rows0, agg_sh.at[dst_v.at[a]], add=True)

            @pl.when(jj < HALF - 1)
            def _():
                pltpu.async_copy(y_hbm.at[src_v.at[a + 2]], rows0, sem_g0)

            pltpu.make_async_copy(y_hbm.at[src_v.at[1]], rows1, sem_g1).wait()
            pltpu.sync_copy(rows1, agg_sh.at[dst_v.at[a + 1]], add=True)

            @pl.when(jj < HALF - 1)
            def _():
                pltpu.async_copy(y_hbm.at[src_v.at[a + 3]], rows1, sem_g1)

            return carry2

        lax.fori_loop(0, HALF, _pair, 0)
        return carry

    lax.fori_loop(0, GROUPS, _group, 0)
    plsc.subcore_barrier()

    for t in range(ROWS_PER_SUB // STAGE):
        r0 = s * ROWS_PER_SUB + t * STAGE
        pltpu.sync_copy(agg_sh.at[pl.ds(r0, STAGE)], stage_v)
        pltpu.sync_copy(stage_v, out_hbm.at[c, pl.ds(r0, STAGE)])


BM = 1000


def _dinv_block(degp_ref):
    deg = degp_ref[0, :, 0:1] + degp_ref[1, :, 0:1] + 1.0
    return lax.rsqrt(deg)


def _mm_scale_body(x_ref, w_ref, degp_ref, y_ref, dinv_ref):
    hw = jnp.dot(x_ref[...], w_ref[...], preferred_element_type=jnp.float32)
    dinv = _dinv_block(degp_ref)
    y_ref[...] = hw * dinv
    dinv_ref[...] = dinv


def _mid_body(aggp_ref, y1_ref, dinv_ref, b_ref, w_ref, y2_ref):
    dinv = dinv_ref[...]
    t = (aggp_ref[0] + aggp_ref[1] + y1_ref[...]) * dinv + b_ref[...]
    h = jnp.maximum(t, 0.0)
    y2_ref[...] = jnp.dot(h, w_ref[...], preferred_element_type=jnp.float32) * dinv


def _out_body(aggp_ref, y2_ref, dinv_ref, b_ref, o_ref):
    dinv = dinv_ref[...]
    o_ref[...] = (aggp_ref[0] + aggp_ref[1] + y2_ref[...]) * dinv + b_ref[...]


_ROW = pl.BlockSpec((BM, D), lambda i: (i, 0))
_FULL = pl.BlockSpec((D, D), lambda i: (0, 0))
_DEGP = pl.BlockSpec((NC, BM, D), lambda i: (0, i, 0))
_AGGP = pl.BlockSpec((NC, BM, D), lambda i: (0, i, 0))
_BIAS = pl.BlockSpec((1, D), lambda i: (0, 0))
_DINV = pl.BlockSpec((BM, 1), lambda i: (i, 0))
_OUT = jax.ShapeDtypeStruct((N_NODES, D), jnp.float32)
_DINV_OUT = jax.ShapeDtypeStruct((N_NODES, 1), jnp.float32)
_GRID = (N_NODES // BM,)

_mm_scale = pl.pallas_call(
    _mm_scale_body, grid=_GRID, out_shape=[_OUT, _DINV_OUT],
    in_specs=[_ROW, _FULL, _DEGP], out_specs=[_ROW, _DINV],
)
_mid = pl.pallas_call(
    _mid_body, grid=_GRID, out_shape=_OUT,
    in_specs=[_AGGP, _ROW, _DINV, _BIAS, _FULL], out_specs=_ROW,
)
_out = pl.pallas_call(
    _out_body, grid=_GRID, out_shape=_OUT,
    in_specs=[_AGGP, _ROW, _DINV, _BIAS], out_specs=_ROW,
)


def kernel(x, edge_index, W1, b1, W2, b2):
    src = edge_index[0].astype(jnp.int32)
    dst = edge_index[1].astype(jnp.int32)
    npadded = NW * EPW - N_EDGES
    spread = jnp.arange(npadded, dtype=jnp.int32) % (NPAD - N_NODES)
    src_r = jnp.concatenate([src, spread]).reshape(NW, NCHUNK, CHUNK)
    dst_r = jnp.concatenate([dst, N_NODES + spread]).reshape(NW, NCHUNK, CHUNK)

    degp = _deg_kernel(dst_r)
    y1, dinv = _mm_scale(x, W1, degp)
    agg1 = _agg_kernel(y1, src_r, dst_r)
    y2 = _mid(agg1, y1, dinv, b1.reshape(1, D), W2)
    agg2 = _agg_kernel(y2, src_r, dst_r)
    return _out(agg2, y2, dinv, b2.reshape(1, D))


# speedup vs baseline: 1.0674x; 1.0242x over previous
import functools

import jax
import jax.numpy as jnp
from jax import lax
from jax.experimental import pallas as pl
from jax.experimental.pallas import tpu as pltpu
from jax.experimental.pallas import tpu_sc as plsc

N_NODES = 10000
N_EDGES = 320000
D = 128

NC, NS, L = 2, 16, 16
NW = NC * NS
NPAD = 10240
EPW = 10240
CHUNK = 128
NCHUNK = EPW // CHUNK
DD = 16
ROWS_PER_SUB = NPAD // NS
STAGE = 32

_MESH = plsc.VectorSubcoreMesh(
    core_axis_name="c", subcore_axis_name="s", num_cores=NC, num_subcores=NS
)


@functools.partial(
    pl.kernel,
    out_type=jax.ShapeDtypeStruct((NC, NPAD, D), jnp.float32),
    mesh=_MESH,
    scratch_types=[
        pltpu.VMEM((NCHUNK, CHUNK), jnp.int32),
        pltpu.VMEM((CHUNK, D), jnp.float32),
        pltpu.VMEM((STAGE, D), jnp.float32),
        pltpu.VMEM_SHARED((NPAD, D), jnp.float32),
        pltpu.SemaphoreType.DMA,
    ],
)
def _deg_kernel(dst_hbm, out_hbm, dst_v, ones_v, stage_v, deg_sh, sems):
    c = lax.axis_index("c")
    s = lax.axis_index("s")
    wid = s * NC + c

    def _zrow(i, carry):
        for k in range(D // L):
            stage_v[i, pl.ds(k * L, L)] = jnp.zeros((L,), jnp.float32)
        return carry

    lax.fori_loop(0, STAGE, _zrow, 0)

    def _orow(i, carry):
        for k in range(D // L):
            ones_v[i, pl.ds(k * L, L)] = jnp.ones((L,), jnp.float32)
        return carry

    lax.fori_loop(0, CHUNK, _orow, 0)
    for t in range(ROWS_PER_SUB // STAGE):
        pltpu.sync_copy(
            stage_v, deg_sh.at[pl.ds(s * ROWS_PER_SUB + t * STAGE, STAGE)]
        )
    plsc.subcore_barrier()

    pltpu.sync_copy(dst_hbm.at[wid], dst_v)

    W = 16

    def _edges(t, carry):
        for k in range(W):
            pltpu.async_copy(ones_v, deg_sh.at[dst_v.at[t * W + k]], sems, add=True)
        for k in range(W):
            pltpu.make_async_copy(ones_v, deg_sh.at[dst_v.at[0]], sems).wait()
        return carry

    lax.fori_loop(0, NCHUNK // W, _edges, 0)
    plsc.subcore_barrier()

    r0 = s * ROWS_PER_SUB
    pltpu.sync_copy(deg_sh.at[pl.ds(r0, ROWS_PER_SUB)],
                    out_hbm.at[c, pl.ds(r0, ROWS_PER_SUB)])


GROUPS = 2
GCHUNK = NCHUNK // GROUPS
HALF = GCHUNK // 2


@functools.partial(
    pl.kernel,
    out_type=jax.ShapeDtypeStruct((NC, NPAD, D), jnp.float32),
    mesh=_MESH,
    scratch_types=[
        pltpu.VMEM((GCHUNK, CHUNK), jnp.int32),
        pltpu.VMEM((GCHUNK, CHUNK), jnp.int32),
        pltpu.VMEM((CHUNK, D), jnp.float32),
        pltpu.VMEM((CHUNK, D), jnp.float32),
        pltpu.VMEM((STAGE, D), jnp.float32),
        pltpu.VMEM_SHARED((NPAD, D), jnp.float32),
        pltpu.SemaphoreType.DMA,
        pltpu.SemaphoreType.DMA,
    ],
)
def _agg_kernel(y_hbm, src_hbm, dst_hbm, out_hbm,
                src_v, dst_v, rows0, rows1, stage_v, agg_sh,
                sem_g0, sem_g1):
    c = lax.axis_index("c")
    s = lax.axis_index("s")
    wid = s * NC + c

    def _zrow(i, carry):
        for k in range(D // L):
            stage_v[i, pl.ds(k * L, L)] = jnp.zeros((L,), jnp.float32)
        return carry

    lax.fori_loop(0, STAGE, _zrow, 0)
    for t in range(ROWS_PER_SUB // STAGE):
        pltpu.sync_copy(
            stage_v, agg_sh.at[pl.ds(s * ROWS_PER_SUB + t * STAGE, STAGE)]
        )
    plsc.subcore_barrier()

    def _group(g, carry):
        pltpu.sync_copy(src_hbm.at[wid, pl.ds(g * GCHUNK, GCHUNK)], src_v)
        pltpu.sync_copy(dst_hbm.at[wid, pl.ds(g * GCHUNK, GCHUNK)], dst_v)
        pltpu.async_copy(y_hbm.at[src_v.at[0]], rows0, sem_g0)
        pltpu.async_copy(y_hbm.at[src_v.at[1]], rows1, sem_g1)

        def _pair(jj, carry2):
            a = 2 * jj
            pltpu.make_async_copy(y_hbm.at[src_v.at[0]], rows0, sem_g0).wait()
            pltpu.sync_copy(rows0, agg_sh.at[dst_v.at[a]], add=True)

            @pl.when(jj < HALF - 1)
            def _():
                pltpu.async_copy(y_hbm.at[src_v.at[a + 2]], rows0, sem_g0)

            pltpu.make_async_copy(y_hbm.at[src_v.at[1]], rows1, sem_g1).wait()
            pltpu.sync_copy(rows1, agg_sh.at[dst_v.at[a + 1]], add=True)

            @pl.when(jj < HALF - 1)
            def _():
                pltpu.async_copy(y_hbm.at[src_v.at[a + 3]], rows1, sem_g1)

            return carry2

        lax.fori_loop(0, HALF, _pair, 0)
        return carry

    lax.fori_loop(0, GROUPS, _group, 0)
    plsc.subcore_barrier()

    r0 = s * ROWS_PER_SUB
    pltpu.sync_copy(agg_sh.at[pl.ds(r0, ROWS_PER_SUB)],
                    out_hbm.at[c, pl.ds(r0, ROWS_PER_SUB)])


BM = 1000


def _dinv_block(degp_ref):
    deg = degp_ref[0, :, 0:1] + degp_ref[1, :, 0:1] + 1.0
    return lax.rsqrt(deg)


def _mm_scale_body(x_ref, w_ref, degp_ref, y_ref, dinv_ref):
    hw = jnp.dot(x_ref[...], w_ref[...], preferred_element_type=jnp.float32)
    dinv = _dinv_block(degp_ref)
    y_ref[...] = hw * dinv
    dinv_ref[...] = dinv


def _mid_body(aggp_ref, y1_ref, dinv_ref, b_ref, w_ref, y2_ref):
    dinv = dinv_ref[...]
    t = (aggp_ref[0] + aggp_ref[1] + y1_ref[...]) * dinv + b_ref[...]
    h = jnp.maximum(t, 0.0)
    y2_ref[...] = jnp.dot(h, w_ref[...], preferred_element_type=jnp.float32) * dinv


def _out_body(aggp_ref, y2_ref, dinv_ref, b_ref, o_ref):
    dinv = dinv_ref[...]
    o_ref[...] = (aggp_ref[0] + aggp_ref[1] + y2_ref[...]) * dinv + b_ref[...]


_ROW = pl.BlockSpec((BM, D), lambda i: (i, 0))
_FULL = pl.BlockSpec((D, D), lambda i: (0, 0))
_DEGP = pl.BlockSpec((NC, BM, D), lambda i: (0, i, 0))
_AGGP = pl.BlockSpec((NC, BM, D), lambda i: (0, i, 0))
_BIAS = pl.BlockSpec((1, D), lambda i: (0, 0))
_DINV = pl.BlockSpec((BM, 1), lambda i: (i, 0))
_OUT = jax.ShapeDtypeStruct((N_NODES, D), jnp.float32)
_DINV_OUT = jax.ShapeDtypeStruct((N_NODES, 1), jnp.float32)
_GRID = (N_NODES // BM,)

_mm_scale = pl.pallas_call(
    _mm_scale_body, grid=_GRID, out_shape=[_OUT, _DINV_OUT],
    in_specs=[_ROW, _FULL, _DEGP], out_specs=[_ROW, _DINV],
)
_mid = pl.pallas_call(
    _mid_body, grid=_GRID, out_shape=_OUT,
    in_specs=[_AGGP, _ROW, _DINV, _BIAS, _FULL], out_specs=_ROW,
)
_out = pl.pallas_call(
    _out_body, grid=_GRID, out_shape=_OUT,
    in_specs=[_AGGP, _ROW, _DINV, _BIAS], out_specs=_ROW,
)


def kernel(x, edge_index, W1, b1, W2, b2):
    src = edge_index[0].astype(jnp.int32)
    dst = edge_index[1].astype(jnp.int32)
    npadded = NW * EPW - N_EDGES
    spread = jnp.arange(npadded, dtype=jnp.int32) % (NPAD - N_NODES)
    src_r = jnp.concatenate([src, spread]).reshape(NW, NCHUNK, CHUNK)
    dst_r = jnp.concatenate([dst, N_NODES + spread]).reshape(NW, NCHUNK, CHUNK)

    degp = _deg_kernel(dst_r)
    y1, dinv = _mm_scale(x, W1, degp)
    agg1 = _agg_kernel(y1, src_r, dst_r)
    y2 = _mid(agg1, y1, dinv, b1.reshape(1, D), W2)
    agg2 = _agg_kernel(y2, src_r, dst_r)
    return _out(agg2, y2, dinv, b2.reshape(1, D))
